# baseline (device time: 33581 ns/iter reference)
import jax
import jax.numpy as jnp
from jax import lax
from jax.experimental import pallas as pl
from jax.experimental.pallas import tpu as pltpu

Y_SIZE = 2


def _body(o_ref, wo_ref, out_ref, wb_ref, part_ref, send_ref, recv_ref,
          send_sem, recv_sem):
    my_x = lax.axis_index("x")
    my_y = lax.axis_index("y")
    my_z = lax.axis_index("z")
    other_y = 1 - my_y
    nbr = (my_x, other_y, my_z)

    barrier_sem = pltpu.get_barrier_semaphore()
    pl.semaphore_signal(barrier_sem, inc=1, device_id=nbr,
                        device_id_type=pl.DeviceIdType.MESH)
    pl.semaphore_wait(barrier_sem, 1)

    wb_ref[...] = wo_ref[...].astype(jnp.bfloat16)

    b_sz, s_half, _ = out_ref.shape

    for b in range(b_sz):
        ob = o_ref[b, pl.ds(other_y * s_half, s_half), :].astype(jnp.bfloat16)
        send_ref[b, :, :] = lax.dot(
            ob, wb_ref[...], preferred_element_type=jnp.float32
        ).astype(jnp.bfloat16)

    rdma = pltpu.make_async_remote_copy(
        src_ref=send_ref,
        dst_ref=recv_ref,
        send_sem=send_sem,
        recv_sem=recv_sem,
        device_id=nbr,
        device_id_type=pl.DeviceIdType.MESH,
    )
    rdma.start()

    for b in range(b_sz):
        ob = o_ref[b, pl.ds(my_y * s_half, s_half), :].astype(jnp.bfloat16)
        part_ref[b, :, :] = lax.dot(
            ob, wb_ref[...], preferred_element_type=jnp.float32
        )

    rdma.wait()
    out_ref[...] = part_ref[...] + recv_ref[...].astype(jnp.float32)


def kernel(O, Wo):
    B, S, H, D = O.shape
    K = H * D
    N = Wo.shape[1]
    s_half = S // Y_SIZE
    O3 = O.reshape(B, S, K)
    return pl.pallas_call(
        _body,
        out_shape=jax.ShapeDtypeStruct((B, s_half, N), jnp.float32),
        in_specs=[
            pl.BlockSpec(memory_space=pltpu.VMEM),
            pl.BlockSpec(memory_space=pltpu.VMEM),
        ],
        out_specs=pl.BlockSpec(memory_space=pltpu.VMEM),
        scratch_shapes=[
            pltpu.VMEM((K, N), jnp.bfloat16),
            pltpu.VMEM((B, s_half, N), jnp.float32),
            pltpu.VMEM((B, s_half, N), jnp.bfloat16),
            pltpu.VMEM((B, s_half, N), jnp.bfloat16),
            pltpu.SemaphoreType.DMA,
            pltpu.SemaphoreType.DMA,
        ],
        compiler_params=pltpu.CompilerParams(collective_id=0),
    )(O3, Wo)


# device time: 32456 ns/iter; 1.0347x vs baseline; 1.0347x over previous
import jax
import jax.numpy as jnp
from jax import lax
from jax.experimental import pallas as pl
from jax.experimental.pallas import tpu as pltpu

Y_SIZE = 2


def _body(o_ref, wo_ref, out_ref, wb_ref, part_ref, send_ref, recv_ref,
          send_sems, recv_sems):
    my_x = lax.axis_index("x")
    my_y = lax.axis_index("y")
    my_z = lax.axis_index("z")
    other_y = 1 - my_y
    nbr = (my_x, other_y, my_z)

    barrier_sem = pltpu.get_barrier_semaphore()
    pl.semaphore_signal(barrier_sem, inc=1, device_id=nbr,
                        device_id_type=pl.DeviceIdType.MESH)
    pl.semaphore_wait(barrier_sem, 1)

    wb_ref[...] = wo_ref[...].astype(jnp.bfloat16)

    b_sz, s_half, _ = out_ref.shape

    def chunk_rdma(b):
        return pltpu.make_async_remote_copy(
            src_ref=send_ref.at[b],
            dst_ref=recv_ref.at[b],
            send_sem=send_sems.at[b],
            recv_sem=recv_sems.at[b],
            device_id=nbr,
            device_id_type=pl.DeviceIdType.MESH,
        )

    for b in range(b_sz):
        ob = o_ref[b, pl.ds(other_y * s_half, s_half), :].astype(jnp.bfloat16)
        send_ref[b, :, :] = lax.dot(
            ob, wb_ref[...], preferred_element_type=jnp.float32
        ).astype(jnp.bfloat16)
        chunk_rdma(b).start()

    for b in range(b_sz):
        ob = o_ref[b, pl.ds(my_y * s_half, s_half), :].astype(jnp.bfloat16)
        part_ref[b, :, :] = lax.dot(
            ob, wb_ref[...], preferred_element_type=jnp.float32
        )

    for b in range(b_sz):
        chunk_rdma(b).wait_recv()
        out_ref[b, :, :] = part_ref[b, :, :] + recv_ref[b, :, :].astype(
            jnp.float32
        )
    for b in range(b_sz):
        chunk_rdma(b).wait_send()


def kernel(O, Wo):
    B, S, H, D = O.shape
    K = H * D
    N = Wo.shape[1]
    s_half = S // Y_SIZE
    O3 = O.reshape(B, S, K)
    return pl.pallas_call(
        _body,
        out_shape=jax.ShapeDtypeStruct((B, s_half, N), jnp.float32),
        in_specs=[
            pl.BlockSpec(memory_space=pltpu.VMEM),
            pl.BlockSpec(memory_space=pltpu.VMEM),
        ],
        out_specs=pl.BlockSpec(memory_space=pltpu.VMEM),
        scratch_shapes=[
            pltpu.VMEM((K, N), jnp.bfloat16),
            pltpu.VMEM((B, s_half, N), jnp.float32),
            pltpu.VMEM((B, s_half, N), jnp.bfloat16),
            pltpu.VMEM((B, s_half, N), jnp.bfloat16),
            pltpu.SemaphoreType.DMA((B,)),
            pltpu.SemaphoreType.DMA((B,)),
        ],
        compiler_params=pltpu.CompilerParams(collective_id=0),
    )(O3, Wo)
